# baseline (speedup 1.0000x reference)
import functools

import jax
import jax.numpy as jnp
from jax import lax
from jax.experimental import pallas as pl
from jax.experimental.pallas import tpu as pltpu
from jax.experimental.pallas import tpu_sc as plsc

LATENT = 32
BATCH = 16384
NROWS = 1000000

NC = 2
NS = 16
NW = NC * NS
B_PER_W = BATCH // NW
CHUNK = 128
NCHUNK = B_PER_W // CHUNK

TBLK = 1024
MLP_BLK = 2048


def _tr_body(t_ref, out_ref):
    x = t_ref[...]
    lane = (pl.program_id(0) * TBLK
            + jax.lax.broadcasted_iota(jnp.int32, (LATENT, TBLK), 1))
    x = jnp.where(lane < NROWS, x, 0.0)
    q = TBLK // 4
    liota = jax.lax.broadcasted_iota(jnp.int32, (TBLK, q), 0)
    qiota = jax.lax.broadcasted_iota(jnp.int32, (TBLK, q), 1)
    for k in range(4):
        sel = jnp.where(liota == 4 * qiota + k, 1.0, 0.0)
        out_ref[:, 32 * k:32 * (k + 1)] = lax.dot_general(
            sel, x, dimension_numbers=(((0,), (1,)), ((), ())),
            preferred_element_type=jnp.float32)


@jax.jit
def _tc_transpose(t):
    grid = (pl.cdiv(NROWS, TBLK),)
    return pl.pallas_call(
        _tr_body,
        grid=grid,
        in_specs=[pl.BlockSpec((LATENT, TBLK), lambda i: (0, i))],
        out_specs=pl.BlockSpec((TBLK // 4, 4 * LATENT), lambda i: (i, 0)),
        out_shape=jax.ShapeDtypeStruct((NROWS // 4, 4 * LATENT), jnp.float32),
    )(t)


def _gather_body(tab, idx_hbm, out, idx_v, rows, sem):
    wid = lax.axis_index("s") * NC + lax.axis_index("c")
    base = wid * B_PER_W
    pltpu.sync_copy(idx_hbm.at[wid], idx_v)
    copies = []
    for j in range(NCHUNK):
        copies.append(pltpu.async_copy(
            tab.at[idx_v.at[j]], rows.at[pl.ds(j * CHUNK, CHUNK)], sem))
    for c in copies:
        c.wait()
    pltpu.sync_copy(rows, out.at[pl.ds(base, B_PER_W)])


@jax.jit
def _sc_gather(tab, idx):
    mesh = plsc.VectorSubcoreMesh(core_axis_name="c", subcore_axis_name="s")
    return pl.kernel(
        _gather_body,
        out_type=jax.ShapeDtypeStruct((BATCH, LATENT), jnp.float32),
        mesh=mesh,
        scratch_types=[
            pltpu.VMEM((NCHUNK, CHUNK), jnp.int32),
            pltpu.VMEM((B_PER_W, LATENT), jnp.float32),
            pltpu.SemaphoreType.DMA,
        ],
        compiler_params=pltpu.CompilerParams(use_tc_tiling_on_sc=False),
    )(tab, idx)


def _mlp_body(sv, av, w1a, w1b, b1, w2, b2, w3, b3, out):
    h = jnp.maximum(
        jnp.dot(sv[...], w1a[...], preferred_element_type=jnp.float32)
        + jnp.dot(av[...], w1b[...], preferred_element_type=jnp.float32)
        + b1[...], 0.0)
    h = jnp.maximum(
        jnp.dot(h, w2[...], preferred_element_type=jnp.float32) + b2[...], 0.0)
    out[...] = jnp.sum(h * w3[...], axis=-1, keepdims=True) + b3[...]


@jax.jit
def _tc_mlp(sv, av, w1a, w1b, b1, w2, b2, w3, b3):
    grid = (BATCH // MLP_BLK,)
    full = lambda shape: pl.BlockSpec(shape, lambda i: (0, 0))
    return pl.pallas_call(
        _mlp_body,
        grid=grid,
        in_specs=[
            pl.BlockSpec((MLP_BLK, LATENT), lambda i: (i, 0)),
            pl.BlockSpec((MLP_BLK, LATENT), lambda i: (i, 0)),
            full((LATENT, 64)),
            full((LATENT, 64)),
            full((1, 64)),
            full((64, LATENT)),
            full((1, LATENT)),
            full((1, LATENT)),
            full((1, 1)),
        ],
        out_specs=pl.BlockSpec((MLP_BLK, 1), lambda i: (i, 0)),
        out_shape=jax.ShapeDtypeStruct((BATCH, 1), jnp.float32),
    )(sv, av, w1a, w1b, b1, w2, b2, w3, b3)


def kernel(inputs, student_table, assessment_table, W1, b1, W2, b2, W3, b3):
    idx_s = inputs[:, 0].reshape(NW, NCHUNK, CHUNK)
    idx_a = inputs[:, 1].reshape(NW, NCHUNK, CHUNK)
    ts = _tc_transpose(student_table.T).reshape(NROWS, LATENT)
    sv = _sc_gather(ts, idx_s)
    ta = _tc_transpose(assessment_table.T).reshape(NROWS, LATENT)
    av = _sc_gather(ta, idx_a)
    return _tc_mlp(
        sv, av,
        W1[:LATENT], W1[LATENT:], b1.reshape(1, 64),
        W2, b2.reshape(1, LATENT),
        W3.reshape(1, LATENT), b3.reshape(1, 1),
    )


# speedup vs baseline: 2.1576x; 2.1576x over previous
import functools

import jax
import jax.numpy as jnp
from jax import lax
from jax.experimental import pallas as pl
from jax.experimental.pallas import tpu as pltpu
from jax.experimental.pallas import tpu_sc as plsc

LATENT = 32
BATCH = 16384
NROWS = 1000000

NC = 2
NS = 16
NW = NC * NS
B_PER_W = BATCH // NW
CHUNK = 128
NCHUNK = B_PER_W // CHUNK

TBLK = 1024
MLP_BLK = 2048


def _tr_body(t_ref, out_ref):
    x = t_ref[...]
    lane = (pl.program_id(0) * TBLK
            + jax.lax.broadcasted_iota(jnp.int32, (LATENT, TBLK), 1))
    x = jnp.where(lane < NROWS, x, 0.0)
    q = TBLK // 4
    liota = jax.lax.broadcasted_iota(jnp.int32, (TBLK, q), 0)
    qiota = jax.lax.broadcasted_iota(jnp.int32, (TBLK, q), 1)
    for k in range(4):
        sel = jnp.where(liota == 4 * qiota + k, 1.0, 0.0)
        out_ref[:, 32 * k:32 * (k + 1)] = lax.dot_general(
            sel, x, dimension_numbers=(((0,), (1,)), ((), ())),
            preferred_element_type=jnp.float32)


@jax.jit
def _tc_transpose(t):
    grid = (pl.cdiv(NROWS, TBLK),)
    return pl.pallas_call(
        _tr_body,
        grid=grid,
        in_specs=[pl.BlockSpec((LATENT, TBLK), lambda i: (0, i))],
        out_specs=pl.BlockSpec((TBLK // 4, 4 * LATENT), lambda i: (i, 0)),
        out_shape=jax.ShapeDtypeStruct((NROWS // 4, 4 * LATENT), jnp.float32),
    )(t)


def _gather_body(tab, idx_hbm, out, idx_v, rows, sem):
    wid = lax.axis_index("s") * NC + lax.axis_index("c")
    base = wid * B_PER_W
    pltpu.sync_copy(idx_hbm.at[wid], idx_v)
    copies = []
    for j in range(NCHUNK):
        copies.append(pltpu.async_copy(
            tab.at[idx_v.at[j]], rows.at[pl.ds(j * CHUNK, CHUNK)], sem))
    for c in copies:
        c.wait()
    pltpu.sync_copy(rows, out.at[pl.ds(base, B_PER_W)])


@jax.jit
def _sc_gather(tab, idx):
    mesh = plsc.VectorSubcoreMesh(core_axis_name="c", subcore_axis_name="s")
    return pl.kernel(
        _gather_body,
        out_type=jax.ShapeDtypeStruct((BATCH, LATENT), jnp.float32),
        mesh=mesh,
        scratch_types=[
            pltpu.VMEM((NCHUNK, CHUNK), jnp.int32),
            pltpu.VMEM((B_PER_W, LATENT), jnp.float32),
            pltpu.SemaphoreType.DMA,
        ],
        compiler_params=pltpu.CompilerParams(use_tc_tiling_on_sc=False),
    )(tab, idx)


def _mlp_body(sv, av, w1a, w1b, b1, w2, b2, w3, b3, out):
    h = jnp.maximum(
        jnp.dot(sv[...], w1a[...], preferred_element_type=jnp.float32)
        + jnp.dot(av[...], w1b[...], preferred_element_type=jnp.float32)
        + b1[...], 0.0)
    h = jnp.maximum(
        jnp.dot(h, w2[...], preferred_element_type=jnp.float32) + b2[...], 0.0)
    out[...] = jnp.sum(h * w3[...], axis=-1, keepdims=True) + b3[...]


@jax.jit
def _tc_mlp(sv, av, w1a, w1b, b1, w2, b2, w3, b3):
    grid = (BATCH // MLP_BLK,)
    full = lambda shape: pl.BlockSpec(shape, lambda i: (0, 0))
    return pl.pallas_call(
        _mlp_body,
        grid=grid,
        in_specs=[
            pl.BlockSpec((MLP_BLK, LATENT), lambda i: (i, 0)),
            pl.BlockSpec((MLP_BLK, LATENT), lambda i: (i, 0)),
            full((LATENT, 64)),
            full((LATENT, 64)),
            full((1, 64)),
            full((64, LATENT)),
            full((1, LATENT)),
            full((1, LATENT)),
            full((1, 1)),
        ],
        out_specs=pl.BlockSpec((MLP_BLK, 1), lambda i: (i, 0)),
        out_shape=jax.ShapeDtypeStruct((BATCH, 1), jnp.float32),
    )(sv, av, w1a, w1b, b1, w2, b2, w3, b3)


def kernel(inputs, student_table, assessment_table, W1, b1, W2, b2, W3, b3):
    idx_s = inputs[:, 0].reshape(NW, NCHUNK, CHUNK)
    idx_a = inputs[:, 1].reshape(NW, NCHUNK, CHUNK)
    sv = _sc_gather(student_table, idx_s)
    av = _sc_gather(assessment_table, idx_a)
    return _tc_mlp(
        sv, av,
        W1[:LATENT], W1[LATENT:], b1.reshape(1, 64),
        W2, b2.reshape(1, LATENT),
        W3.reshape(1, LATENT), b3.reshape(1, 1),
    )
